# baseline (speedup 1.0000x reference)
import functools

import jax
import jax.numpy as jnp
from jax import lax
from jax.experimental import pallas as pl
from jax.experimental.pallas import tpu as pltpu
from jax.experimental.pallas import tpu_sc as plsc

_N = 10000
_A = 32
_E = 128
_P625 = 625
_TILE = 1000
_GRID = _N // _TILE
_HTAB = _P625 + _TILE


def _sc_gather_body(anchor_hbm, emb_hbm, a32_hbm, idx_v, rows_v, sem):
    c = lax.axis_index("c")
    s = lax.axis_index("s")
    wid = s * 2 + c

    @pl.when(wid == 0)
    def _():
        pltpu.sync_copy(anchor_hbm, idx_v)
        pltpu.async_copy(emb_hbm.at[idx_v], rows_v, sem).wait()
        pltpu.sync_copy(rows_v, a32_hbm)


def _sc_gather(anchor_set_id, embeds):
    sc_kernel = functools.partial(
        pl.kernel,
        mesh=plsc.VectorSubcoreMesh(core_axis_name="c", subcore_axis_name="s"),
        out_type=jax.ShapeDtypeStruct((_A, _E), jnp.float32),
        scratch_types=[
            pltpu.VMEM((_A,), jnp.int32),
            pltpu.VMEM((_A, _E), jnp.float32),
            pltpu.SemaphoreType.DMA,
        ],
    )(_sc_gather_body)
    return sc_kernel(anchor_set_id, embeds)


def _tc1_body(d_ref, e_ref, w_ref, b_ref, h_ref, dt_ref):
    W2 = w_ref[:, _E:]
    B2 = e_ref[...].reshape(_P625, 16, _E).sum(axis=1)
    r_io = lax.broadcasted_iota(jnp.int32, (_HTAB, _P625), 0)
    j_io = lax.broadcasted_iota(jnp.int32, (_HTAB, _P625), 1)
    perm = (jnp.equal((2 * r_io) % _P625, j_io)
            | jnp.equal((2 * r_io + 1) % _P625, j_io)).astype(jnp.float32)
    S2 = jnp.dot(perm, B2, preferred_element_type=jnp.float32)
    h_ref[...] = lax.dot_general(
        S2, W2, (((1,), (1,)), ((), ())),
        preferred_element_type=jnp.float32) * (1.0 / _A) + b_ref[...]
    dt_ref[...] = jnp.transpose(d_ref[...], (1, 0))


def _tc1(dists, embeds, W, b2d):
    return pl.pallas_call(
        _tc1_body,
        out_shape=[
            jax.ShapeDtypeStruct((_HTAB, _E), jnp.float32),
            jax.ShapeDtypeStruct((_N, _A), jnp.float32),
        ],
    )(dists, embeds, W, b2d)


def _tc2_body(a32_ref, dt_ref, h_ref, w_ref, out_ref, p_scr):
    t = pl.program_id(0)

    @pl.when(t == 0)
    def _init():
        p_scr[...] = lax.dot_general(
            a32_ref[...], w_ref[:, :_E], (((1,), (1,)), ((), ())),
            preferred_element_type=jnp.float32) * (1.0 / _A)

    s_t = (t * _TILE) % _P625
    out_ref[...] = (jnp.dot(dt_ref[...], p_scr[...],
                            preferred_element_type=jnp.float32)
                    + h_ref[pl.ds(s_t, _TILE), :])


def _tc2(a32, dt, htab, W):
    return pl.pallas_call(
        _tc2_body,
        grid=(_GRID,),
        in_specs=[
            pl.BlockSpec((_A, _E), lambda t: (0, 0)),
            pl.BlockSpec((_TILE, _A), lambda t: (t, 0)),
            pl.BlockSpec((_HTAB, _E), lambda t: (0, 0)),
            pl.BlockSpec((_E, 2 * _E), lambda t: (0, 0)),
        ],
        out_specs=pl.BlockSpec((_TILE, _E), lambda t: (t, 0)),
        out_shape=jax.ShapeDtypeStruct((_N, _E), jnp.float32),
        scratch_shapes=[pltpu.VMEM((_A, _E), jnp.float32)],
    )(a32, dt, htab, W)


def kernel(anchor_set_id, dists_array, embeds, W, b):
    a32 = _sc_gather(anchor_set_id, embeds)
    b2d = b.reshape(1, _E)
    htab, dt = _tc1(dists_array, embeds, W, b2d)
    return _tc2(a32, dt, htab, W)


# speedup vs baseline: 1.0398x; 1.0398x over previous
import functools

import jax
import jax.numpy as jnp
from jax import lax
from jax.experimental import pallas as pl
from jax.experimental.pallas import tpu as pltpu
from jax.experimental.pallas import tpu_sc as plsc

_N = 10000
_A = 32
_E = 128
_P625 = 625
_TILE = 1000
_GRID = _N // _TILE
_HTAB = _P625 + _TILE


def _sc_gather_body(anchor_hbm, emb_hbm, a32_hbm, idx_v, rows_v, sem):
    c = lax.axis_index("c")
    s = lax.axis_index("s")
    wid = s * 2 + c

    @pl.when(wid == 0)
    def _():
        pltpu.sync_copy(anchor_hbm, idx_v)
        pltpu.async_copy(emb_hbm.at[idx_v], rows_v, sem).wait()
        pltpu.sync_copy(rows_v, a32_hbm)


def _sc_gather(anchor_set_id, embeds):
    sc_kernel = functools.partial(
        pl.kernel,
        mesh=plsc.VectorSubcoreMesh(core_axis_name="c", subcore_axis_name="s"),
        out_type=jax.ShapeDtypeStruct((_A, _E), jnp.float32),
        scratch_types=[
            pltpu.VMEM((_A,), jnp.int32),
            pltpu.VMEM((_A, _E), jnp.float32),
            pltpu.SemaphoreType.DMA,
        ],
    )(_sc_gather_body)
    return sc_kernel(anchor_set_id, embeds)


def _tc_body(a32_ref, d_ref, e_ref, w_ref, b_ref, out_ref,
             p_scr, h_scr, dt_scr):
    t = pl.program_id(0)

    @pl.when(t == 0)
    def _init():
        W1 = w_ref[:, :_E]
        W2 = w_ref[:, _E:]
        B2 = e_ref[...].reshape(_P625, 16, _E).sum(axis=1)
        r_io = lax.broadcasted_iota(jnp.int32, (_HTAB, _P625), 0)
        j_io = lax.broadcasted_iota(jnp.int32, (_HTAB, _P625), 1)
        perm = (jnp.equal((2 * r_io) % _P625, j_io)
                | jnp.equal((2 * r_io + 1) % _P625, j_io)).astype(jnp.float32)
        S2 = jnp.dot(perm, B2, preferred_element_type=jnp.float32)
        h_scr[...] = lax.dot_general(
            S2, W2, (((1,), (1,)), ((), ())),
            preferred_element_type=jnp.float32) * (1.0 / _A) + b_ref[...]
        p_scr[...] = lax.dot_general(
            a32_ref[...], W1, (((1,), (1,)), ((), ())),
            preferred_element_type=jnp.float32) * (1.0 / _A)
        dt_scr[...] = jnp.transpose(d_ref[...], (1, 0))

    s_t = (t * _TILE) % _P625
    out_ref[...] = (jnp.dot(dt_scr[pl.ds(t * _TILE, _TILE), :], p_scr[...],
                            preferred_element_type=jnp.float32)
                    + h_scr[pl.ds(s_t, _TILE), :])


def kernel(anchor_set_id, dists_array, embeds, W, b):
    a32 = _sc_gather(anchor_set_id, embeds)
    b2d = b.reshape(1, _E)
    return pl.pallas_call(
        _tc_body,
        grid=(_GRID,),
        in_specs=[
            pl.BlockSpec((_A, _E), lambda t: (0, 0)),
            pl.BlockSpec((_A, _N), lambda t: (0, 0)),
            pl.BlockSpec((_N, _E), lambda t: (0, 0)),
            pl.BlockSpec((_E, 2 * _E), lambda t: (0, 0)),
            pl.BlockSpec((1, _E), lambda t: (0, 0)),
        ],
        out_specs=pl.BlockSpec((_TILE, _E), lambda t: (t, 0)),
        out_shape=jax.ShapeDtypeStruct((_N, _E), jnp.float32),
        scratch_shapes=[
            pltpu.VMEM((_A, _E), jnp.float32),
            pltpu.VMEM((_HTAB, _E), jnp.float32),
            pltpu.VMEM((_N, _A), jnp.float32),
        ],
    )(a32, dists_array, embeds, W, b2d)
